# baseline (speedup 1.0000x reference)
import functools

import jax
import jax.numpy as jnp
from jax.experimental import pallas as pl
from jax.experimental.pallas import tpu as pltpu

N_EMBED = 4096
NUM_EXPERTS = 64
TOP_K = 8
N_TOKENS = 8192

TOKEN_BLOCK = 256


def _router_kernel(
    x_ref, wr_ref, wn_ref, br_ref, bn_ref, eps_ref, router_ref, idx_ref, wcat_ref
):
    @pl.when(pl.program_id(0) == 0)
    def _pack_weights():
        wcat_ref[:NUM_EXPERTS, :] = wr_ref[...].astype(jnp.bfloat16)
        wcat_ref[NUM_EXPERTS:, :] = wn_ref[...].astype(jnp.bfloat16)

    x = x_ref[...].astype(jnp.bfloat16)
    logits_cat = jax.lax.dot_general(
        x,
        wcat_ref[...],
        dimension_numbers=(((1,), (1,)), ((), ())),
        preferred_element_type=jnp.float32,
    )

    logits = logits_cat[:, :NUM_EXPERTS] + br_ref[...]
    noise_logits = logits_cat[:, NUM_EXPERTS:] + bn_ref[...]
    noisy = logits + eps_ref[...] * jax.nn.softplus(noise_logits)

    noisy_t = noisy.T
    rowf = jax.lax.broadcasted_iota(jnp.int32, noisy_t.shape, 0).astype(jnp.float32)
    vals = noisy_t
    neg_inf = jnp.float32(-jnp.inf)
    top1 = None
    idx_rows = []
    for k in range(TOP_K):
        m = jnp.max(vals, axis=0, keepdims=True)
        if k == 0:
            top1 = m
        idx = jnp.min(
            jnp.where(vals == m, rowf, jnp.float32(NUM_EXPERTS)),
            axis=0,
            keepdims=True,
        )
        idx_rows.append(idx)
        vals = jnp.where(rowf == idx, neg_inf, vals)

    idx_t = jnp.concatenate(idx_rows, axis=0)
    idx_ref[...] = idx_t.T.astype(jnp.int32)

    selected = vals == neg_inf
    e = jnp.where(selected, jnp.exp(noisy_t - top1), 0.0)
    denom = jnp.sum(e, axis=0, keepdims=True)
    router_ref[...] = (e / denom).T


def kernel(mh_output, W_route, b_route, W_noise, b_noise, noise_eps):
    n_tokens = mh_output.shape[0]
    grid = (n_tokens // TOKEN_BLOCK,)

    router_out, idx_out = pl.pallas_call(
        _router_kernel,
        grid=grid,
        in_specs=[
            pl.BlockSpec((TOKEN_BLOCK, N_EMBED), lambda i: (i, 0)),
            pl.BlockSpec((NUM_EXPERTS, N_EMBED), lambda i: (0, 0)),
            pl.BlockSpec((NUM_EXPERTS, N_EMBED), lambda i: (0, 0)),
            pl.BlockSpec((1, NUM_EXPERTS), lambda i: (0, 0)),
            pl.BlockSpec((1, NUM_EXPERTS), lambda i: (0, 0)),
            pl.BlockSpec((TOKEN_BLOCK, NUM_EXPERTS), lambda i: (i, 0)),
        ],
        out_specs=[
            pl.BlockSpec((TOKEN_BLOCK, NUM_EXPERTS), lambda i: (i, 0)),
            pl.BlockSpec((TOKEN_BLOCK, TOP_K), lambda i: (i, 0)),
        ],
        out_shape=[
            jax.ShapeDtypeStruct((n_tokens, NUM_EXPERTS), jnp.float32),
            jax.ShapeDtypeStruct((n_tokens, TOP_K), jnp.int32),
        ],
        scratch_shapes=[pltpu.VMEM((2 * NUM_EXPERTS, N_EMBED), jnp.bfloat16)],
        compiler_params=pltpu.CompilerParams(
            dimension_semantics=("parallel",),
        ),
    )(mh_output, W_route, W_noise, b_route[None, :], b_noise[None, :], noise_eps)

    return (router_out, idx_out)


# speedup vs baseline: 1.1767x; 1.1767x over previous
import functools

import jax
import jax.numpy as jnp
from jax.experimental import pallas as pl
from jax.experimental.pallas import tpu as pltpu

N_EMBED = 4096
NUM_EXPERTS = 64
TOP_K = 8
N_TOKENS = 8192

TOKEN_BLOCK = 512


def _router_kernel(
    x_ref, wr_ref, wn_ref, br_ref, bn_ref, eps_ref, router_ref, idx_ref, wcat_ref
):
    @pl.when(pl.program_id(0) == 0)
    def _pack_weights():
        wcat_ref[:NUM_EXPERTS, :] = wr_ref[...].astype(jnp.bfloat16)
        wcat_ref[NUM_EXPERTS:, :] = wn_ref[...].astype(jnp.bfloat16)

    x = x_ref[...].astype(jnp.bfloat16)
    logits_cat = jax.lax.dot_general(
        x,
        wcat_ref[...],
        dimension_numbers=(((1,), (1,)), ((), ())),
        preferred_element_type=jnp.float32,
    )

    logits = logits_cat[:, :NUM_EXPERTS] + br_ref[...]
    noise_logits = logits_cat[:, NUM_EXPERTS:] + bn_ref[...]
    noisy = logits + eps_ref[...] * jax.nn.softplus(noise_logits)

    noisy_t = noisy.T
    rowf = jax.lax.broadcasted_iota(jnp.int32, noisy_t.shape, 0).astype(jnp.float32)
    vals = noisy_t
    neg_inf = jnp.float32(-jnp.inf)
    top1 = None
    idx_rows = []
    for k in range(TOP_K):
        m = jnp.max(vals, axis=0, keepdims=True)
        if k == 0:
            top1 = m
        idx = jnp.min(
            jnp.where(vals == m, rowf, jnp.float32(NUM_EXPERTS)),
            axis=0,
            keepdims=True,
        )
        idx_rows.append(idx)
        vals = jnp.where(rowf == idx, neg_inf, vals)

    idx_t = jnp.concatenate(idx_rows, axis=0)
    idx_ref[...] = idx_t.T.astype(jnp.int32)

    selected = vals == neg_inf
    e = jnp.where(selected, jnp.exp(noisy_t - top1), 0.0)
    denom = jnp.sum(e, axis=0, keepdims=True)
    router_ref[...] = (e / denom).T


def kernel(mh_output, W_route, b_route, W_noise, b_noise, noise_eps):
    n_tokens = mh_output.shape[0]
    grid = (n_tokens // TOKEN_BLOCK,)

    router_out, idx_out = pl.pallas_call(
        _router_kernel,
        grid=grid,
        in_specs=[
            pl.BlockSpec((TOKEN_BLOCK, N_EMBED), lambda i: (i, 0)),
            pl.BlockSpec((NUM_EXPERTS, N_EMBED), lambda i: (0, 0)),
            pl.BlockSpec((NUM_EXPERTS, N_EMBED), lambda i: (0, 0)),
            pl.BlockSpec((1, NUM_EXPERTS), lambda i: (0, 0)),
            pl.BlockSpec((1, NUM_EXPERTS), lambda i: (0, 0)),
            pl.BlockSpec((TOKEN_BLOCK, NUM_EXPERTS), lambda i: (i, 0)),
        ],
        out_specs=[
            pl.BlockSpec((TOKEN_BLOCK, NUM_EXPERTS), lambda i: (i, 0)),
            pl.BlockSpec((TOKEN_BLOCK, TOP_K), lambda i: (i, 0)),
        ],
        out_shape=[
            jax.ShapeDtypeStruct((n_tokens, NUM_EXPERTS), jnp.float32),
            jax.ShapeDtypeStruct((n_tokens, TOP_K), jnp.int32),
        ],
        scratch_shapes=[pltpu.VMEM((2 * NUM_EXPERTS, N_EMBED), jnp.bfloat16)],
        compiler_params=pltpu.CompilerParams(
            dimension_semantics=("parallel",),
        ),
    )(mh_output, W_route, W_noise, b_route[None, :], b_noise[None, :], noise_eps)

    return (router_out, idx_out)


# speedup vs baseline: 1.2237x; 1.0400x over previous
import functools

import jax
import jax.numpy as jnp
from jax.experimental import pallas as pl
from jax.experimental.pallas import tpu as pltpu

N_EMBED = 4096
NUM_EXPERTS = 64
TOP_K = 8
N_TOKENS = 8192

TOKEN_BLOCK = 1024


def _router_kernel(
    x_ref, wr_ref, wn_ref, br_ref, bn_ref, eps_ref, router_ref, idx_ref, wcat_ref
):
    @pl.when(pl.program_id(0) == 0)
    def _pack_weights():
        wcat_ref[:NUM_EXPERTS, :] = wr_ref[...].astype(jnp.bfloat16)
        wcat_ref[NUM_EXPERTS:, :] = wn_ref[...].astype(jnp.bfloat16)

    x = x_ref[...].astype(jnp.bfloat16)
    logits_cat = jax.lax.dot_general(
        x,
        wcat_ref[...],
        dimension_numbers=(((1,), (1,)), ((), ())),
        preferred_element_type=jnp.float32,
    )

    logits = logits_cat[:, :NUM_EXPERTS] + br_ref[...]
    noise_logits = logits_cat[:, NUM_EXPERTS:] + bn_ref[...]
    noisy = logits + eps_ref[...] * jax.nn.softplus(noise_logits)

    noisy_t = noisy.T
    rowf = jax.lax.broadcasted_iota(jnp.int32, noisy_t.shape, 0).astype(jnp.float32)
    vals = noisy_t
    neg_inf = jnp.float32(-jnp.inf)
    top1 = None
    idx_rows = []
    for k in range(TOP_K):
        m = jnp.max(vals, axis=0, keepdims=True)
        if k == 0:
            top1 = m
        idx = jnp.min(
            jnp.where(vals == m, rowf, jnp.float32(NUM_EXPERTS)),
            axis=0,
            keepdims=True,
        )
        idx_rows.append(idx)
        vals = jnp.where(rowf == idx, neg_inf, vals)

    idx_t = jnp.concatenate(idx_rows, axis=0)
    idx_ref[...] = idx_t.T.astype(jnp.int32)

    selected = vals == neg_inf
    e = jnp.where(selected, jnp.exp(noisy_t - top1), 0.0)
    denom = jnp.sum(e, axis=0, keepdims=True)
    router_ref[...] = (e / denom).T


def kernel(mh_output, W_route, b_route, W_noise, b_noise, noise_eps):
    n_tokens = mh_output.shape[0]
    grid = (n_tokens // TOKEN_BLOCK,)

    router_out, idx_out = pl.pallas_call(
        _router_kernel,
        grid=grid,
        in_specs=[
            pl.BlockSpec((TOKEN_BLOCK, N_EMBED), lambda i: (i, 0)),
            pl.BlockSpec((NUM_EXPERTS, N_EMBED), lambda i: (0, 0)),
            pl.BlockSpec((NUM_EXPERTS, N_EMBED), lambda i: (0, 0)),
            pl.BlockSpec((1, NUM_EXPERTS), lambda i: (0, 0)),
            pl.BlockSpec((1, NUM_EXPERTS), lambda i: (0, 0)),
            pl.BlockSpec((TOKEN_BLOCK, NUM_EXPERTS), lambda i: (i, 0)),
        ],
        out_specs=[
            pl.BlockSpec((TOKEN_BLOCK, NUM_EXPERTS), lambda i: (i, 0)),
            pl.BlockSpec((TOKEN_BLOCK, TOP_K), lambda i: (i, 0)),
        ],
        out_shape=[
            jax.ShapeDtypeStruct((n_tokens, NUM_EXPERTS), jnp.float32),
            jax.ShapeDtypeStruct((n_tokens, TOP_K), jnp.int32),
        ],
        scratch_shapes=[pltpu.VMEM((2 * NUM_EXPERTS, N_EMBED), jnp.bfloat16)],
        compiler_params=pltpu.CompilerParams(
            dimension_semantics=("parallel",),
        ),
    )(mh_output, W_route, W_noise, b_route[None, :], b_noise[None, :], noise_eps)

    return (router_out, idx_out)
